# baseline (speedup 1.0000x reference)
import functools

import jax
import jax.numpy as jnp
from jax import lax
from jax.experimental import pallas as pl
from jax.experimental.pallas import tpu as pltpu
from jax.experimental.pallas import tpu_sc as plsc

N = 10000
E = 160000
D = 256

NC = 2
NS = 16
NW = NC * NS
CPW = D // NW

CB = 640
NCHUNK = E // CB
GSUB = 128
NG = CB // GSUB

ROWBLK = 1000


def _mm_dual_body(x_ref, wp_ref, bp_ref, ws_ref, bs_ref, hp_ref, s_ref):
    xb = x_ref[...]
    hp = jnp.dot(xb, wp_ref[...], preferred_element_type=jnp.float32)
    hp_ref[...] = jnp.maximum(hp + bp_ref[...], 0.0)
    s = jnp.dot(xb, ws_ref[...], preferred_element_type=jnp.float32)
    s_ref[...] = s + bs_ref[...]


def _mm_dual(h, WpT, bp, WsT, bs):
    return pl.pallas_call(
        _mm_dual_body,
        grid=(N // ROWBLK,),
        in_specs=[
            pl.BlockSpec((ROWBLK, D), lambda i: (i, 0)),
            pl.BlockSpec((D, D), lambda i: (0, 0)),
            pl.BlockSpec((1, D), lambda i: (0, 0)),
            pl.BlockSpec((D, D), lambda i: (0, 0)),
            pl.BlockSpec((1, D), lambda i: (0, 0)),
        ],
        out_specs=[
            pl.BlockSpec((ROWBLK, D), lambda i: (i, 0)),
            pl.BlockSpec((ROWBLK, D), lambda i: (i, 0)),
        ],
        out_shape=[jax.ShapeDtypeStruct((N, D), jnp.float32)] * 2,
    )(h, WpT, bp.reshape(1, D), WsT, bs.reshape(1, D))


def _mm_out_body(act, s_ref, agg_ref, wn_ref, o_ref):
    o = s_ref[...] + jnp.dot(agg_ref[...], wn_ref[...],
                             preferred_element_type=jnp.float32)
    if act:
        o = jnp.tanh(o)
    o_ref[...] = o


def _mm_out(s, agg, WnT, act):
    return pl.pallas_call(
        functools.partial(_mm_out_body, act),
        grid=(N // ROWBLK,),
        in_specs=[
            pl.BlockSpec((ROWBLK, D), lambda i: (i, 0)),
            pl.BlockSpec((ROWBLK, D), lambda i: (i, 0)),
            pl.BlockSpec((D, D), lambda i: (0, 0)),
        ],
        out_specs=pl.BlockSpec((ROWBLK, D), lambda i: (i, 0)),
        out_shape=jax.ShapeDtypeStruct((N, D), jnp.float32),
    )(s, agg, WnT)


Q0 = 3334
Q1 = 3334
Q2 = N - Q0 - Q1
B1 = Q0 * CPW
B2 = (Q0 + Q1) * CPW
ACC_TOT = N * CPW


def _segmax_body(hp8_hbm, gidx_hbm, dst8_hbm, out_hbm,
                 gidx_a, dst8_a, rows_a, gidx_b, dst8_b, rows_b,
                 acc0, acc1, acc2, stream0, stream1, stream2,
                 isem_a, isem_b, gsem_a, gsem_b):
    w = lax.axis_index("s") * NC + lax.axis_index("c")

    iota = lax.iota(jnp.int32, 16)
    colpat = jnp.bitwise_and(iota, 7)
    pairsel = jnp.right_shift(iota, 3)
    perm8 = jnp.bitwise_xor(iota, 8)
    wvec = jnp.full((16,), 0, jnp.int32) + w
    zeros16 = jnp.zeros((16,), jnp.float32)

    for acc, q in ((acc0, Q0), (acc1, Q1), (acc2, Q2)):
        def zbody(i, carry, acc=acc):
            acc[pl.ds(i * 16, 16)] = zeros16
            return carry

        lax.fori_loop(0, (q * CPW) // 16, zbody, 0)

    def fire_idx(c, gidx_v, dst8_v, isem):
        e0 = c * CB
        cp1 = pltpu.async_copy(gidx_hbm.at[pl.ds(e0, CB)], gidx_v, isem)
        cp2 = pltpu.async_copy(dst8_hbm.at[pl.ds(e0, CB)], dst8_v, isem)
        return cp1, cp2

    def addw_fire_rows(gidx_v, rows_v, gsem):
        for i in range(CB // 16):
            sl = pl.ds(i * 16, 16)
            gidx_v[sl] = gidx_v[sl] + wvec
        return [
            pltpu.async_copy(hp8_hbm.at[gidx_v.at[pl.ds(j * GSUB, GSUB)]],
                             rows_v.at[pl.ds(j * GSUB, GSUB)], gsem)
            for j in range(NG)
        ]

    bcast15 = jnp.full((16,), 15, jnp.int32)

    def bclast(x):
        return x.at[bcast15].get(mode="promise_in_bounds")

    def pair_loop(dst8_v, rows_v):
        def grp_body(i, offs):
            off0, off1, off2 = offs
            d8 = dst8_v[pl.ds(i * 16, 16)]
            ep17 = lax.shift_left(iota + i * 16, 17)
            c0 = d8 < B1
            c1 = d8 < B2
            m1 = c1 & (~c0)
            m2 = ~c1
            new = []
            for stream, acc_base, mq, off in ((stream0, 0, c0, off0),
                                              (stream1, B1, m1, off1),
                                              (stream2, B2, m2, off2)):
                entry = (d8 - acc_base) | ep17
                cum = plsc.cumsum(mq.astype(jnp.int32))
                tidx = off + cum - 1
                plsc.store_scatter(stream, [tidx], entry, mask=mq)
                new.append(off + bclast(cum))
            return tuple(new)

        zoff = jnp.zeros((16,), jnp.int32)
        off0, off1, off2 = lax.fori_loop(0, CB // 16, grp_body,
                                         (zoff, zoff, zoff), unroll=4)

        maxlen = jnp.maximum(jnp.maximum(off0, off1), off2)
        nit = (jnp.max(maxlen) + 3) >> 2

        def step(pb, stream, acc, lenv):
            epat = pairsel + pb
            mt = epat < lenv
            entry = plsc.load_gather(stream, [epat])
            entry = jnp.where(mt, entry, 0x1FFFF)
            d8c = entry & 0x1FFFF
            epos = lax.shift_right_logical(entry, 17)
            fidx = d8c + colpat
            r = plsc.load_gather(rows_v, [epos, colpat])
            a = plsc.load_gather(acc, [fidx], mask=mt)
            m = jnp.maximum(a, r)
            dswp = d8c.at[perm8].get(mode="promise_in_bounds",
                                     unique_indices=True)
            msw = m.at[perm8].get(mode="promise_in_bounds",
                                  unique_indices=True)
            msel = jnp.where(d8c == dswp, jnp.maximum(m, msw), m)
            plsc.store_scatter(acc, [fidx], msel, mask=mt)

        def pp_body(t, carry2):
            for s in range(2):
                pb = (t * 2 + s) * 2
                step(pb, stream0, acc0, off0)
                step(pb, stream1, acc1, off1)
                step(pb, stream2, acc2, off2)
            return carry2

        lax.fori_loop(0, nit, pp_body, 0)

    last = NCHUNK - 1

    ia = fire_idx(0, gidx_a, dst8_a, isem_a)
    ib = fire_idx(1, gidx_b, dst8_b, isem_b)
    ia[0].wait()
    ia[1].wait()
    ga = addw_fire_rows(gidx_a, rows_a, gsem_a)

    def body(cc, carry):
        ca = 2 * cc
        pltpu.make_async_copy(gidx_hbm.at[pl.ds(0, CB)], gidx_b, isem_b).wait()
        pltpu.make_async_copy(dst8_hbm.at[pl.ds(0, CB)], dst8_b, isem_b).wait()
        addw_fire_rows(gidx_b, rows_b, gsem_b)
        for j in range(NG):
            pltpu.make_async_copy(
                hp8_hbm.at[gidx_a.at[pl.ds(j * GSUB, GSUB)]],
                rows_a.at[pl.ds(j * GSUB, GSUB)], gsem_a).wait()
        pair_loop(dst8_a, rows_a)
        nca = jnp.minimum(ca + 2, last)
        fire_idx(nca, gidx_a, dst8_a, isem_a)
        for j in range(NG):
            pltpu.make_async_copy(
                hp8_hbm.at[gidx_b.at[pl.ds(j * GSUB, GSUB)]],
                rows_b.at[pl.ds(j * GSUB, GSUB)], gsem_b).wait()
        pair_loop(dst8_b, rows_b)
        pltpu.make_async_copy(gidx_hbm.at[pl.ds(0, CB)], gidx_a, isem_a).wait()
        pltpu.make_async_copy(dst8_hbm.at[pl.ds(0, CB)], dst8_a, isem_a).wait()
        addw_fire_rows(gidx_a, rows_a, gsem_a)
        ncb = jnp.minimum(ca + 3, last)
        fire_idx(ncb, gidx_b, dst8_b, isem_b)
        return carry

    lax.fori_loop(0, NCHUNK // 2, body, 0)
    pltpu.make_async_copy(gidx_hbm.at[pl.ds(0, CB)], gidx_b, isem_b).wait()
    pltpu.make_async_copy(dst8_hbm.at[pl.ds(0, CB)], dst8_b, isem_b).wait()
    for j in range(NG):
        pltpu.make_async_copy(
            hp8_hbm.at[gidx_a.at[pl.ds(j * GSUB, GSUB)]],
            rows_a.at[pl.ds(j * GSUB, GSUB)], gsem_a).wait()
    ob = w * ACC_TOT
    pltpu.sync_copy(acc0, out_hbm.at[pl.ds(ob, B1)])
    pltpu.sync_copy(acc1, out_hbm.at[pl.ds(ob + B1, B2 - B1)])
    pltpu.sync_copy(acc2, out_hbm.at[pl.ds(ob + B2, ACC_TOT - B2)])


_segmax = pl.kernel(
    _segmax_body,
    out_type=jax.ShapeDtypeStruct((NW * N * CPW,), jnp.float32),
    mesh=plsc.VectorSubcoreMesh(core_axis_name="c", subcore_axis_name="s",
                                num_cores=NC, num_subcores=NS),
    scratch_types=[
        pltpu.VMEM((CB,), jnp.int32),
        pltpu.VMEM((CB,), jnp.int32),
        pltpu.VMEM((CB, CPW), jnp.float32),
        pltpu.VMEM((CB,), jnp.int32),
        pltpu.VMEM((CB,), jnp.int32),
        pltpu.VMEM((CB, CPW), jnp.float32),
        pltpu.VMEM((Q0 * CPW,), jnp.float32),
        pltpu.VMEM((Q1 * CPW,), jnp.float32),
        pltpu.VMEM((Q2 * CPW,), jnp.float32),
        pltpu.VMEM((CB + 8,), jnp.int32),
        pltpu.VMEM((CB + 8,), jnp.int32),
        pltpu.VMEM((CB + 8,), jnp.int32),
        pltpu.SemaphoreType.DMA,
        pltpu.SemaphoreType.DMA,
        pltpu.SemaphoreType.DMA,
        pltpu.SemaphoreType.DMA,
    ],
    compiler_params=pltpu.CompilerParams(needs_layout_passes=False,
                                         use_tc_tiling_on_sc=False),
)


def _sage_layer(h, gidx32, dst8, WpT, bp, WsT, WnT, bs, act):
    hp, s = _mm_dual(h, WpT, bp, WsT, bs)
    hp8 = hp.reshape(N * NW, CPW)
    agg32 = _segmax(hp8, gidx32, dst8)
    agg = agg32.reshape(NW, N, CPW).transpose(1, 0, 2).reshape(N, D)
    return _mm_out(s, agg, WnT, act)


def kernel(x, edge_index, W_pool1, b_pool1, W_self1, W_neigh1, bias1,
           W_pool2, b_pool2, W_self2, W_neigh2, bias2):
    src = edge_index[0]
    dst = edge_index[1]
    gidx32 = src * NW
    dst8 = dst * CPW
    h = _sage_layer(x, gidx32, dst8, W_pool1.T, b_pool1, W_self1.T,
                    W_neigh1.T, bias1, True)
    h = _sage_layer(h, gidx32, dst8, W_pool2.T, b_pool2, W_self2.T,
                    W_neigh2.T, bias2, False)
    return h


# speedup vs baseline: 1.4212x; 1.4212x over previous
import functools

import jax
import jax.numpy as jnp
from jax import lax
from jax.experimental import pallas as pl
from jax.experimental.pallas import tpu as pltpu
from jax.experimental.pallas import tpu_sc as plsc

N = 10000
E = 160000
D = 256

NC = 2
NS = 16
NW = NC * NS
CPW = D // NW

CB = 640
NCHUNK = E // CB
GSUB = 128
NG = CB // GSUB

ROWBLK = 1000


def _mm_dual_body(x_ref, wp_ref, bp_ref, ws_ref, bs_ref, hp_ref, s_ref):
    xb = x_ref[...]
    hp = jnp.dot(xb, wp_ref[...], preferred_element_type=jnp.float32)
    hp_ref[...] = jnp.maximum(hp + bp_ref[...], 0.0)
    s = jnp.dot(xb, ws_ref[...], preferred_element_type=jnp.float32)
    s_ref[...] = s + bs_ref[...]


def _mm_dual(h, WpT, bp, WsT, bs):
    return pl.pallas_call(
        _mm_dual_body,
        grid=(N // ROWBLK,),
        in_specs=[
            pl.BlockSpec((ROWBLK, D), lambda i: (i, 0)),
            pl.BlockSpec((D, D), lambda i: (0, 0)),
            pl.BlockSpec((1, D), lambda i: (0, 0)),
            pl.BlockSpec((D, D), lambda i: (0, 0)),
            pl.BlockSpec((1, D), lambda i: (0, 0)),
        ],
        out_specs=[
            pl.BlockSpec((ROWBLK, D), lambda i: (i, 0)),
            pl.BlockSpec((ROWBLK, D), lambda i: (i, 0)),
        ],
        out_shape=[jax.ShapeDtypeStruct((N, D), jnp.float32)] * 2,
    )(h, WpT, bp.reshape(1, D), WsT, bs.reshape(1, D))


def _mm_out_body(act, s_ref, agg_ref, wn_ref, o_ref):
    o = s_ref[...] + jnp.dot(agg_ref[...], wn_ref[...],
                             preferred_element_type=jnp.float32)
    if act:
        o = jnp.tanh(o)
    o_ref[...] = o


def _mm_out(s, agg, WnT, act):
    return pl.pallas_call(
        functools.partial(_mm_out_body, act),
        grid=(N // ROWBLK,),
        in_specs=[
            pl.BlockSpec((ROWBLK, D), lambda i: (i, 0)),
            pl.BlockSpec((ROWBLK, D), lambda i: (i, 0)),
            pl.BlockSpec((D, D), lambda i: (0, 0)),
        ],
        out_specs=pl.BlockSpec((ROWBLK, D), lambda i: (i, 0)),
        out_shape=jax.ShapeDtypeStruct((N, D), jnp.float32),
    )(s, agg, WnT)


Q0 = 3334
Q1 = 3334
Q2 = N - Q0 - Q1
B1 = Q0 * CPW
B2 = (Q0 + Q1) * CPW
ACC_TOT = N * CPW


def _segmax_body(hp8_hbm, gidx_hbm, dst8_hbm, out_hbm,
                 gidx_a, dst8_a, rows_a, gidx_b, dst8_b, rows_b,
                 acc0, acc1, acc2, stream0, stream1, stream2,
                 isem_a, isem_b, gsem_a, gsem_b):
    w = lax.axis_index("s") * NC + lax.axis_index("c")

    iota = lax.iota(jnp.int32, 16)
    colpat = jnp.bitwise_and(iota, 7)
    pairsel = jnp.right_shift(iota, 3)
    perm8 = jnp.bitwise_xor(iota, 8)
    wvec = jnp.full((16,), 0, jnp.int32) + w
    zeros16 = jnp.zeros((16,), jnp.float32)

    for acc, q in ((acc0, Q0), (acc1, Q1), (acc2, Q2)):
        def zbody(i, carry, acc=acc):
            acc[pl.ds(i * 16, 16)] = zeros16
            return carry

        lax.fori_loop(0, (q * CPW) // 16, zbody, 0)

    def fire_idx(c, gidx_v, dst8_v, isem):
        e0 = c * CB
        cp1 = pltpu.async_copy(gidx_hbm.at[pl.ds(e0, CB)], gidx_v, isem)
        cp2 = pltpu.async_copy(dst8_hbm.at[pl.ds(e0, CB)], dst8_v, isem)
        return cp1, cp2

    def addw_fire_rows(gidx_v, rows_v, gsem):
        for i in range(CB // 16):
            sl = pl.ds(i * 16, 16)
            gidx_v[sl] = gidx_v[sl] + wvec
        return [
            pltpu.async_copy(hp8_hbm.at[gidx_v.at[pl.ds(j * GSUB, GSUB)]],
                             rows_v.at[pl.ds(j * GSUB, GSUB)], gsem)
            for j in range(NG)
        ]

    bcast15 = jnp.full((16,), 15, jnp.int32)

    def bclast(x):
        return x.at[bcast15].get(mode="promise_in_bounds")

    def pair_loop(dst8_v, rows_v):
        def grp_body(i, offs):
            off0, off1, off2 = offs
            d8 = dst8_v[pl.ds(i * 16, 16)]
            ep17 = lax.shift_left(iota + i * 16, 17)
            c0 = d8 < B1
            c1 = d8 < B2
            m1 = c1 & (~c0)
            m2 = ~c1
            new = []
            for stream, acc_base, mq, off in ((stream0, 0, c0, off0),
                                              (stream1, B1, m1, off1),
                                              (stream2, B2, m2, off2)):
                entry = (d8 - acc_base) | ep17
                cum = plsc.cumsum(mq.astype(jnp.int32))
                tidx = off + cum - 1
                plsc.store_scatter(stream, [tidx], entry, mask=mq)
                new.append(off + bclast(cum))
            return tuple(new)

        zoff = jnp.zeros((16,), jnp.int32)
        off0, off1, off2 = lax.fori_loop(0, CB // 16, grp_body,
                                         (zoff, zoff, zoff), unroll=4)

        streams = (stream0, stream1, stream2)
        accs = (acc0, acc1, acc2)
        lens = (off0, off1, off2)
        maxlen = jnp.maximum(jnp.maximum(off0, off1), off2)
        nit = (jnp.max(maxlen) + 1) >> 1

        def fetch(si, t):
            epat = pairsel + t * 2
            mt = epat < lens[si]
            entry = plsc.load_gather(streams[si], [epat])
            entry = jnp.where(mt, entry, 0x1FFFF)
            d8c = entry & 0x1FFFF
            epos = lax.shift_right_logical(entry, 17)
            r = plsc.load_gather(rows_v, [epos, colpat])
            return d8c, r

        pre = [fetch(si, 0) for si in range(3)]
        carry0 = (pre[0][0], pre[0][1], pre[1][0], pre[1][1],
                  pre[2][0], pre[2][1])

        def pp_body(t, carry2):
            cur = ((carry2[0], carry2[1]), (carry2[2], carry2[3]),
                   (carry2[4], carry2[5]))
            nxt = [fetch(si, t + 1) for si in range(3)]
            epat = pairsel + t * 2
            for si in range(3):
                d8c, r = cur[si]
                mt = epat < lens[si]
                fidx = d8c + colpat
                a = plsc.load_gather(accs[si], [fidx], mask=mt)
                m = jnp.maximum(a, r)
                dswp = d8c.at[perm8].get(mode="promise_in_bounds",
                                         unique_indices=True)
                msw = m.at[perm8].get(mode="promise_in_bounds",
                                      unique_indices=True)
                msel = jnp.where(d8c == dswp, jnp.maximum(m, msw), m)
                plsc.store_scatter(accs[si], [fidx], msel, mask=mt)
            return (nxt[0][0], nxt[0][1], nxt[1][0], nxt[1][1],
                    nxt[2][0], nxt[2][1])

        lax.fori_loop(0, nit, pp_body, carry0)

    last = NCHUNK - 1

    ia = fire_idx(0, gidx_a, dst8_a, isem_a)
    ib = fire_idx(1, gidx_b, dst8_b, isem_b)
    ia[0].wait()
    ia[1].wait()
    ga = addw_fire_rows(gidx_a, rows_a, gsem_a)

    def body(cc, carry):
        ca = 2 * cc
        pltpu.make_async_copy(gidx_hbm.at[pl.ds(0, CB)], gidx_b, isem_b).wait()
        pltpu.make_async_copy(dst8_hbm.at[pl.ds(0, CB)], dst8_b, isem_b).wait()
        addw_fire_rows(gidx_b, rows_b, gsem_b)
        for j in range(NG):
            pltpu.make_async_copy(
                hp8_hbm.at[gidx_a.at[pl.ds(j * GSUB, GSUB)]],
                rows_a.at[pl.ds(j * GSUB, GSUB)], gsem_a).wait()
        pair_loop(dst8_a, rows_a)
        nca = jnp.minimum(ca + 2, last)
        fire_idx(nca, gidx_a, dst8_a, isem_a)
        for j in range(NG):
            pltpu.make_async_copy(
                hp8_hbm.at[gidx_b.at[pl.ds(j * GSUB, GSUB)]],
                rows_b.at[pl.ds(j * GSUB, GSUB)], gsem_b).wait()
        pair_loop(dst8_b, rows_b)
        pltpu.make_async_copy(gidx_hbm.at[pl.ds(0, CB)], gidx_a, isem_a).wait()
        pltpu.make_async_copy(dst8_hbm.at[pl.ds(0, CB)], dst8_a, isem_a).wait()
        addw_fire_rows(gidx_a, rows_a, gsem_a)
        ncb = jnp.minimum(ca + 3, last)
        fire_idx(ncb, gidx_b, dst8_b, isem_b)
        return carry

    lax.fori_loop(0, NCHUNK // 2, body, 0)
    pltpu.make_async_copy(gidx_hbm.at[pl.ds(0, CB)], gidx_b, isem_b).wait()
    pltpu.make_async_copy(dst8_hbm.at[pl.ds(0, CB)], dst8_b, isem_b).wait()
    for j in range(NG):
        pltpu.make_async_copy(
            hp8_hbm.at[gidx_a.at[pl.ds(j * GSUB, GSUB)]],
            rows_a.at[pl.ds(j * GSUB, GSUB)], gsem_a).wait()
    ob = w * ACC_TOT
    pltpu.sync_copy(acc0, out_hbm.at[pl.ds(ob, B1)])
    pltpu.sync_copy(acc1, out_hbm.at[pl.ds(ob + B1, B2 - B1)])
    pltpu.sync_copy(acc2, out_hbm.at[pl.ds(ob + B2, ACC_TOT - B2)])


_segmax = pl.kernel(
    _segmax_body,
    out_type=jax.ShapeDtypeStruct((NW * N * CPW,), jnp.float32),
    mesh=plsc.VectorSubcoreMesh(core_axis_name="c", subcore_axis_name="s",
                                num_cores=NC, num_subcores=NS),
    scratch_types=[
        pltpu.VMEM((CB,), jnp.int32),
        pltpu.VMEM((CB,), jnp.int32),
        pltpu.VMEM((CB, CPW), jnp.float32),
        pltpu.VMEM((CB,), jnp.int32),
        pltpu.VMEM((CB,), jnp.int32),
        pltpu.VMEM((CB, CPW), jnp.float32),
        pltpu.VMEM((Q0 * CPW,), jnp.float32),
        pltpu.VMEM((Q1 * CPW,), jnp.float32),
        pltpu.VMEM((Q2 * CPW,), jnp.float32),
        pltpu.VMEM((CB + 8,), jnp.int32),
        pltpu.VMEM((CB + 8,), jnp.int32),
        pltpu.VMEM((CB + 8,), jnp.int32),
        pltpu.SemaphoreType.DMA,
        pltpu.SemaphoreType.DMA,
        pltpu.SemaphoreType.DMA,
        pltpu.SemaphoreType.DMA,
    ],
    compiler_params=pltpu.CompilerParams(needs_layout_passes=False,
                                         use_tc_tiling_on_sc=False),
)


def _sage_layer(h, gidx32, dst8, WpT, bp, WsT, WnT, bs, act):
    hp, s = _mm_dual(h, WpT, bp, WsT, bs)
    hp8 = hp.reshape(N * NW, CPW)
    agg32 = _segmax(hp8, gidx32, dst8)
    agg = agg32.reshape(NW, N, CPW).transpose(1, 0, 2).reshape(N, D)
    return _mm_out(s, agg, WnT, act)


def kernel(x, edge_index, W_pool1, b_pool1, W_self1, W_neigh1, bias1,
           W_pool2, b_pool2, W_self2, W_neigh2, bias2):
    src = edge_index[0]
    dst = edge_index[1]
    gidx32 = src * NW
    dst8 = dst * CPW
    h = _sage_layer(x, gidx32, dst8, W_pool1.T, b_pool1, W_self1.T,
                    W_neigh1.T, bias1, True)
    h = _sage_layer(h, gidx32, dst8, W_pool2.T, b_pool2, W_self2.T,
                    W_neigh2.T, bias2, False)
    return h


# speedup vs baseline: 1.5995x; 1.1255x over previous
import functools

import jax
import jax.numpy as jnp
from jax import lax
from jax.experimental import pallas as pl
from jax.experimental.pallas import tpu as pltpu
from jax.experimental.pallas import tpu_sc as plsc

N = 10000
E = 160000
D = 256

NC = 2
NS = 16
NW = NC * NS
CPW = D // NW

CB = 640
NCHUNK = E // CB
GSUB = 128
NG = CB // GSUB

ROWBLK = 1000


def _mm_dual_body(x_ref, wp_ref, bp_ref, ws_ref, bs_ref, hp_ref, s_ref):
    xb = x_ref[...]
    hp = jnp.dot(xb, wp_ref[...], preferred_element_type=jnp.float32)
    hp_ref[...] = jnp.maximum(hp + bp_ref[...], 0.0)
    s = jnp.dot(xb, ws_ref[...], preferred_element_type=jnp.float32)
    s_ref[...] = s + bs_ref[...]


def _mm_dual(h, WpT, bp, WsT, bs):
    return pl.pallas_call(
        _mm_dual_body,
        grid=(N // ROWBLK,),
        in_specs=[
            pl.BlockSpec((ROWBLK, D), lambda i: (i, 0)),
            pl.BlockSpec((D, D), lambda i: (0, 0)),
            pl.BlockSpec((1, D), lambda i: (0, 0)),
            pl.BlockSpec((D, D), lambda i: (0, 0)),
            pl.BlockSpec((1, D), lambda i: (0, 0)),
        ],
        out_specs=[
            pl.BlockSpec((ROWBLK, D), lambda i: (i, 0)),
            pl.BlockSpec((ROWBLK, D), lambda i: (i, 0)),
        ],
        out_shape=[jax.ShapeDtypeStruct((N, D), jnp.float32)] * 2,
    )(h, WpT, bp.reshape(1, D), WsT, bs.reshape(1, D))


def _mm_out_body(act, s_ref, agg_ref, wn_ref, o_ref):
    o = s_ref[...] + jnp.dot(agg_ref[...], wn_ref[...],
                             preferred_element_type=jnp.float32)
    if act:
        o = jnp.tanh(o)
    o_ref[...] = o


def _mm_out(s, agg, WnT, act):
    return pl.pallas_call(
        functools.partial(_mm_out_body, act),
        grid=(N // ROWBLK,),
        in_specs=[
            pl.BlockSpec((ROWBLK, D), lambda i: (i, 0)),
            pl.BlockSpec((ROWBLK, D), lambda i: (i, 0)),
            pl.BlockSpec((D, D), lambda i: (0, 0)),
        ],
        out_specs=pl.BlockSpec((ROWBLK, D), lambda i: (i, 0)),
        out_shape=jax.ShapeDtypeStruct((N, D), jnp.float32),
    )(s, agg, WnT)


Q0 = 3334
Q1 = 3334
Q2 = N - Q0 - Q1
B1 = Q0 * CPW
B2 = (Q0 + Q1) * CPW
ACC_TOT = N * CPW


def _segmax_body(hp8_hbm, gidx_hbm, dst8_hbm, out_hbm,
                 gidx_a, dst8_a, rows_a, gidx_b, dst8_b, rows_b,
                 acc0, acc1, acc2, stream0, stream1, stream2,
                 isem_a, isem_b, gsem_a, gsem_b):
    w = lax.axis_index("s") * NC + lax.axis_index("c")

    iota = lax.iota(jnp.int32, 16)
    colpat = jnp.bitwise_and(iota, 7)
    pairsel = jnp.right_shift(iota, 3)
    perm8 = jnp.bitwise_xor(iota, 8)
    wvec = jnp.full((16,), 0, jnp.int32) + w
    zeros16 = jnp.zeros((16,), jnp.float32)

    for acc, q in ((acc0, Q0), (acc1, Q1), (acc2, Q2)):
        def zbody(i, carry, acc=acc):
            acc[pl.ds(i * 16, 16)] = zeros16
            return carry

        lax.fori_loop(0, (q * CPW) // 16, zbody, 0)

    def fire_idx(c, gidx_v, dst8_v, isem):
        e0 = c * CB
        cp1 = pltpu.async_copy(gidx_hbm.at[pl.ds(e0, CB)], gidx_v, isem)
        cp2 = pltpu.async_copy(dst8_hbm.at[pl.ds(e0, CB)], dst8_v, isem)
        return cp1, cp2

    def addw_fire_rows(gidx_v, rows_v, gsem):
        for i in range(CB // 16):
            sl = pl.ds(i * 16, 16)
            gidx_v[sl] = gidx_v[sl] + wvec
        return [
            pltpu.async_copy(hp8_hbm.at[gidx_v.at[pl.ds(j * GSUB, GSUB)]],
                             rows_v.at[pl.ds(j * GSUB, GSUB)], gsem)
            for j in range(NG)
        ]

    bcast15 = jnp.full((16,), 15, jnp.int32)

    def bclast(x):
        return x.at[bcast15].get(mode="promise_in_bounds")

    def pair_loop(dst8_v, rows_v):
        def grp_body(i, offs):
            off0, off1, off2 = offs
            d8 = dst8_v[pl.ds(i * 16, 16)]
            ep17 = lax.shift_left(iota + i * 16, 17)
            c0 = d8 < B1
            c1 = d8 < B2
            m1 = c1 & (~c0)
            m2 = ~c1
            new = []
            for stream, acc_base, mq, off in ((stream0, 0, c0, off0),
                                              (stream1, B1, m1, off1),
                                              (stream2, B2, m2, off2)):
                entry = (d8 - acc_base) | ep17
                cum = plsc.cumsum(mq.astype(jnp.int32))
                tidx = off + cum - 1
                plsc.store_scatter(stream, [tidx], entry, mask=mq)
                new.append(off + bclast(cum))
            return tuple(new)

        zoff = jnp.zeros((16,), jnp.int32)
        off0, off1, off2 = lax.fori_loop(0, CB // 16, grp_body,
                                         (zoff, zoff, zoff), unroll=4)

        streams = (stream0, stream1, stream2)
        accs = (acc0, acc1, acc2)
        lens = (off0, off1, off2)
        maxlen = jnp.maximum(jnp.maximum(off0, off1), off2)
        nit = (jnp.max(maxlen) + 1) >> 1

        def fetch(si, t):
            epat = pairsel + t * 2
            mt = epat < lens[si]
            entry = plsc.load_gather(streams[si], [epat])
            entry = jnp.where(mt, entry, 0x1FFFF)
            d8c = entry & 0x1FFFF
            epos = lax.shift_right_logical(entry, 17)
            r = plsc.load_gather(rows_v, [epos, colpat])
            dswp = d8c.at[perm8].get(mode="promise_in_bounds",
                                     unique_indices=True)
            rswp = r.at[perm8].get(mode="promise_in_bounds",
                                   unique_indices=True)
            rpre = jnp.where(d8c == dswp, jnp.maximum(r, rswp), r)
            return d8c + colpat, rpre

        def rmw(si, t, fidx, rpre):
            mt = (pairsel + t * 2) < lens[si]
            a = plsc.load_gather(accs[si], [fidx], mask=mt)
            plsc.store_scatter(accs[si], [fidx], jnp.maximum(a, rpre),
                               mask=mt)

        pre = [fetch(si, 0) for si in range(3)]
        carry0 = (pre[0][0], pre[0][1], pre[1][0], pre[1][1],
                  pre[2][0], pre[2][1])
        nit2 = (nit + 1) >> 1

        def pp_body(t2, carry2):
            t = t2 * 2
            cur = ((carry2[0], carry2[1]), (carry2[2], carry2[3]),
                   (carry2[4], carry2[5]))
            mid = [fetch(si, t + 1) for si in range(3)]
            for si in range(3):
                rmw(si, t, *cur[si])
            nxt = [fetch(si, t + 2) for si in range(3)]
            for si in range(3):
                rmw(si, t + 1, *mid[si])
            return (nxt[0][0], nxt[0][1], nxt[1][0], nxt[1][1],
                    nxt[2][0], nxt[2][1])

        lax.fori_loop(0, nit2, pp_body, carry0)

    last = NCHUNK - 1

    ia = fire_idx(0, gidx_a, dst8_a, isem_a)
    ib = fire_idx(1, gidx_b, dst8_b, isem_b)
    ia[0].wait()
    ia[1].wait()
    ga = addw_fire_rows(gidx_a, rows_a, gsem_a)

    def body(cc, carry):
        ca = 2 * cc
        pltpu.make_async_copy(gidx_hbm.at[pl.ds(0, CB)], gidx_b, isem_b).wait()
        pltpu.make_async_copy(dst8_hbm.at[pl.ds(0, CB)], dst8_b, isem_b).wait()
        addw_fire_rows(gidx_b, rows_b, gsem_b)
        for j in range(NG):
            pltpu.make_async_copy(
                hp8_hbm.at[gidx_a.at[pl.ds(j * GSUB, GSUB)]],
                rows_a.at[pl.ds(j * GSUB, GSUB)], gsem_a).wait()
        pair_loop(dst8_a, rows_a)
        nca = jnp.minimum(ca + 2, last)
        fire_idx(nca, gidx_a, dst8_a, isem_a)
        for j in range(NG):
            pltpu.make_async_copy(
                hp8_hbm.at[gidx_b.at[pl.ds(j * GSUB, GSUB)]],
                rows_b.at[pl.ds(j * GSUB, GSUB)], gsem_b).wait()
        pair_loop(dst8_b, rows_b)
        pltpu.make_async_copy(gidx_hbm.at[pl.ds(0, CB)], gidx_a, isem_a).wait()
        pltpu.make_async_copy(dst8_hbm.at[pl.ds(0, CB)], dst8_a, isem_a).wait()
        addw_fire_rows(gidx_a, rows_a, gsem_a)
        ncb = jnp.minimum(ca + 3, last)
        fire_idx(ncb, gidx_b, dst8_b, isem_b)
        return carry

    lax.fori_loop(0, NCHUNK // 2, body, 0)
    pltpu.make_async_copy(gidx_hbm.at[pl.ds(0, CB)], gidx_b, isem_b).wait()
    pltpu.make_async_copy(dst8_hbm.at[pl.ds(0, CB)], dst8_b, isem_b).wait()
    for j in range(NG):
        pltpu.make_async_copy(
            hp8_hbm.at[gidx_a.at[pl.ds(j * GSUB, GSUB)]],
            rows_a.at[pl.ds(j * GSUB, GSUB)], gsem_a).wait()
    ob = w * ACC_TOT
    pltpu.sync_copy(acc0, out_hbm.at[pl.ds(ob, B1)])
    pltpu.sync_copy(acc1, out_hbm.at[pl.ds(ob + B1, B2 - B1)])
    pltpu.sync_copy(acc2, out_hbm.at[pl.ds(ob + B2, ACC_TOT - B2)])


_segmax = pl.kernel(
    _segmax_body,
    out_type=jax.ShapeDtypeStruct((NW * N * CPW,), jnp.float32),
    mesh=plsc.VectorSubcoreMesh(core_axis_name="c", subcore_axis_name="s",
                                num_cores=NC, num_subcores=NS),
    scratch_types=[
        pltpu.VMEM((CB,), jnp.int32),
        pltpu.VMEM((CB,), jnp.int32),
        pltpu.VMEM((CB, CPW), jnp.float32),
        pltpu.VMEM((CB,), jnp.int32),
        pltpu.VMEM((CB,), jnp.int32),
        pltpu.VMEM((CB, CPW), jnp.float32),
        pltpu.VMEM((Q0 * CPW,), jnp.float32),
        pltpu.VMEM((Q1 * CPW,), jnp.float32),
        pltpu.VMEM((Q2 * CPW,), jnp.float32),
        pltpu.VMEM((CB + 8,), jnp.int32),
        pltpu.VMEM((CB + 8,), jnp.int32),
        pltpu.VMEM((CB + 8,), jnp.int32),
        pltpu.SemaphoreType.DMA,
        pltpu.SemaphoreType.DMA,
        pltpu.SemaphoreType.DMA,
        pltpu.SemaphoreType.DMA,
    ],
    compiler_params=pltpu.CompilerParams(needs_layout_passes=False,
                                         use_tc_tiling_on_sc=False),
)


def _sage_layer(h, gidx32, dst8, WpT, bp, WsT, WnT, bs, act):
    hp, s = _mm_dual(h, WpT, bp, WsT, bs)
    hp8 = hp.reshape(N * NW, CPW)
    agg32 = _segmax(hp8, gidx32, dst8)
    agg = agg32.reshape(NW, N, CPW).transpose(1, 0, 2).reshape(N, D)
    return _mm_out(s, agg, WnT, act)


def kernel(x, edge_index, W_pool1, b_pool1, W_self1, W_neigh1, bias1,
           W_pool2, b_pool2, W_self2, W_neigh2, bias2):
    src = edge_index[0]
    dst = edge_index[1]
    gidx32 = src * NW
    dst8 = dst * CPW
    h = _sage_layer(x, gidx32, dst8, W_pool1.T, b_pool1, W_self1.T,
                    W_neigh1.T, bias1, True)
    h = _sage_layer(h, gidx32, dst8, W_pool2.T, b_pool2, W_self2.T,
                    W_neigh2.T, bias2, False)
    return h


# speedup vs baseline: 1.7654x; 1.1037x over previous
import functools

import jax
import jax.numpy as jnp
from jax import lax
from jax.experimental import pallas as pl
from jax.experimental.pallas import tpu as pltpu
from jax.experimental.pallas import tpu_sc as plsc

N = 10000
E = 160000
D = 256

NC = 2
NS = 16
NW = NC * NS
CPW = D // NW

CB = 640
NCHUNK = E // CB
GSUB = 128
NG = CB // GSUB

ROWBLK = 1000


def _mm_dual_body(x_ref, wp_ref, bp_ref, ws_ref, bs_ref, hp_ref, s_ref):
    xb = x_ref[...]
    hp = jnp.dot(xb, wp_ref[...], preferred_element_type=jnp.float32)
    hp_ref[...] = jnp.maximum(hp + bp_ref[...], 0.0)
    s = jnp.dot(xb, ws_ref[...], preferred_element_type=jnp.float32)
    s_ref[...] = s + bs_ref[...]


def _mm_dual(h, WpT, bp, WsT, bs):
    return pl.pallas_call(
        _mm_dual_body,
        grid=(N // ROWBLK,),
        in_specs=[
            pl.BlockSpec((ROWBLK, D), lambda i: (i, 0)),
            pl.BlockSpec((D, D), lambda i: (0, 0)),
            pl.BlockSpec((1, D), lambda i: (0, 0)),
            pl.BlockSpec((D, D), lambda i: (0, 0)),
            pl.BlockSpec((1, D), lambda i: (0, 0)),
        ],
        out_specs=[
            pl.BlockSpec((ROWBLK, D), lambda i: (i, 0)),
            pl.BlockSpec((ROWBLK, D), lambda i: (i, 0)),
        ],
        out_shape=[jax.ShapeDtypeStruct((N, D), jnp.float32)] * 2,
    )(h, WpT, bp.reshape(1, D), WsT, bs.reshape(1, D))


def _mm_out_body(act, s_ref, agg_ref, wn_ref, o_ref):
    o = s_ref[...] + jnp.dot(agg_ref[...], wn_ref[...],
                             preferred_element_type=jnp.float32)
    if act:
        o = jnp.tanh(o)
    o_ref[...] = o


def _mm_out(s, agg, WnT, act):
    return pl.pallas_call(
        functools.partial(_mm_out_body, act),
        grid=(N // ROWBLK,),
        in_specs=[
            pl.BlockSpec((ROWBLK, D), lambda i: (i, 0)),
            pl.BlockSpec((ROWBLK, D), lambda i: (i, 0)),
            pl.BlockSpec((D, D), lambda i: (0, 0)),
        ],
        out_specs=pl.BlockSpec((ROWBLK, D), lambda i: (i, 0)),
        out_shape=jax.ShapeDtypeStruct((N, D), jnp.float32),
    )(s, agg, WnT)


Q0 = 3334
Q1 = 3334
Q2 = N - Q0 - Q1
B1 = Q0 * CPW
B2 = (Q0 + Q1) * CPW
ACC_TOT = N * CPW


def _segmax_body(hp8_hbm, gidx_hbm, dst8_hbm, out_hbm,
                 gidx_a, dst8_a, rows_a, gidx_b, dst8_b, rows_b,
                 acc0, acc1, acc2, stream0, stream1, stream2,
                 isem_a, isem_b, gsem_a, gsem_b):
    w = lax.axis_index("s") * NC + lax.axis_index("c")

    iota = lax.iota(jnp.int32, 16)
    colpat = jnp.bitwise_and(iota, 7)
    pairsel = jnp.right_shift(iota, 3)
    perm8 = jnp.bitwise_xor(iota, 8)
    wvec = jnp.full((16,), 0, jnp.int32) + w
    zeros16 = jnp.zeros((16,), jnp.float32)

    for acc, q in ((acc0, Q0), (acc1, Q1), (acc2, Q2)):
        def zbody(i, carry, acc=acc):
            acc[pl.ds(i * 16, 16)] = zeros16
            return carry

        lax.fori_loop(0, (q * CPW) // 16, zbody, 0)

    def fire_idx(c, gidx_v, dst8_v, isem):
        e0 = c * CB
        cp1 = pltpu.async_copy(gidx_hbm.at[pl.ds(e0, CB)], gidx_v, isem)
        cp2 = pltpu.async_copy(dst8_hbm.at[pl.ds(e0, CB)], dst8_v, isem)
        return cp1, cp2

    def addw_fire_rows(gidx_v, rows_v, gsem):
        for i in range(CB // 16):
            sl = pl.ds(i * 16, 16)
            gidx_v[sl] = gidx_v[sl] + wvec
        return [
            pltpu.async_copy(hp8_hbm.at[gidx_v.at[pl.ds(j * GSUB, GSUB)]],
                             rows_v.at[pl.ds(j * GSUB, GSUB)], gsem)
            for j in range(NG)
        ]

    bcast15 = jnp.full((16,), 15, jnp.int32)

    def bclast(x):
        return x.at[bcast15].get(mode="promise_in_bounds")

    def pair_loop(dst8_v, rows_v, after_compact=None):
        def grp_body(i, offs):
            off0, off1, off2 = offs
            d8 = dst8_v[pl.ds(i * 16, 16)]
            ep17 = lax.shift_left(iota + i * 16, 17)
            c0 = d8 < B1
            c1 = d8 < B2
            m1 = c1 & (~c0)
            m2 = ~c1
            new = []
            for stream, acc_base, mq, off in ((stream0, 0, c0, off0),
                                              (stream1, B1, m1, off1),
                                              (stream2, B2, m2, off2)):
                entry = (d8 - acc_base) | ep17
                cum = plsc.cumsum(mq.astype(jnp.int32))
                tidx = off + cum - 1
                plsc.store_scatter(stream, [tidx], entry, mask=mq)
                new.append(off + bclast(cum))
            return tuple(new)

        zoff = jnp.zeros((16,), jnp.int32)
        off0, off1, off2 = lax.fori_loop(0, CB // 16, grp_body,
                                         (zoff, zoff, zoff), unroll=4)
        if after_compact is not None:
            after_compact()

        streams = (stream0, stream1, stream2)
        accs = (acc0, acc1, acc2)
        lens = (off0, off1, off2)
        maxlen = jnp.maximum(jnp.maximum(off0, off1), off2)
        nit = (jnp.max(maxlen) + 1) >> 1

        def fetch(si, t):
            epat = pairsel + t * 2
            mt = epat < lens[si]
            entry = plsc.load_gather(streams[si], [epat])
            entry = jnp.where(mt, entry, 0x1FFFF)
            d8c = entry & 0x1FFFF
            epos = lax.shift_right_logical(entry, 17)
            r = plsc.load_gather(rows_v, [epos, colpat])
            dswp = d8c.at[perm8].get(mode="promise_in_bounds",
                                     unique_indices=True)
            rswp = r.at[perm8].get(mode="promise_in_bounds",
                                   unique_indices=True)
            rpre = jnp.where(d8c == dswp, jnp.maximum(r, rswp), r)
            return d8c + colpat, rpre

        def rmw(si, t, fidx, rpre):
            mt = (pairsel + t * 2) < lens[si]
            a = plsc.load_gather(accs[si], [fidx], mask=mt)
            plsc.store_scatter(accs[si], [fidx], jnp.maximum(a, rpre),
                               mask=mt)

        pre = [fetch(si, 0) for si in range(3)]
        carry0 = (pre[0][0], pre[0][1], pre[1][0], pre[1][1],
                  pre[2][0], pre[2][1])
        nit2 = (nit + 1) >> 1

        def pp_body(t2, carry2):
            t = t2 * 2
            cur = ((carry2[0], carry2[1]), (carry2[2], carry2[3]),
                   (carry2[4], carry2[5]))
            mid = [fetch(si, t + 1) for si in range(3)]
            for si in range(3):
                rmw(si, t, *cur[si])
            nxt = [fetch(si, t + 2) for si in range(3)]
            for si in range(3):
                rmw(si, t + 1, *mid[si])
            return (nxt[0][0], nxt[0][1], nxt[1][0], nxt[1][1],
                    nxt[2][0], nxt[2][1])

        lax.fori_loop(0, nit2, pp_body, carry0)

    last = NCHUNK - 1

    ia = fire_idx(0, gidx_a, dst8_a, isem_a)
    ib = fire_idx(1, gidx_b, dst8_b, isem_b)
    ia[0].wait()
    ia[1].wait()
    ga = addw_fire_rows(gidx_a, rows_a, gsem_a)

    def body(cc, carry):
        ca = 2 * cc
        nca = jnp.minimum(ca + 2, last)
        pltpu.make_async_copy(gidx_hbm.at[pl.ds(0, CB)], gidx_b, isem_b).wait()
        pltpu.make_async_copy(dst8_hbm.at[pl.ds(0, CB)], dst8_b, isem_b).wait()
        addw_fire_rows(gidx_b, rows_b, gsem_b)
        for j in range(NG):
            pltpu.make_async_copy(
                hp8_hbm.at[gidx_a.at[pl.ds(j * GSUB, GSUB)]],
                rows_a.at[pl.ds(j * GSUB, GSUB)], gsem_a).wait()
        e0a = nca * CB
        pltpu.async_copy(gidx_hbm.at[pl.ds(e0a, CB)], gidx_a, isem_a)

        def refill_dst8_a():
            pltpu.async_copy(dst8_hbm.at[pl.ds(e0a, CB)], dst8_a, isem_a)

        pair_loop(dst8_a, rows_a, after_compact=refill_dst8_a)
        for j in range(NG):
            pltpu.make_async_copy(
                hp8_hbm.at[gidx_b.at[pl.ds(j * GSUB, GSUB)]],
                rows_b.at[pl.ds(j * GSUB, GSUB)], gsem_b).wait()
        pltpu.make_async_copy(gidx_hbm.at[pl.ds(0, CB)], gidx_a, isem_a).wait()
        pltpu.make_async_copy(dst8_hbm.at[pl.ds(0, CB)], dst8_a, isem_a).wait()
        addw_fire_rows(gidx_a, rows_a, gsem_a)
        pair_loop(dst8_b, rows_b)
        ncb = jnp.minimum(ca + 3, last)
        fire_idx(ncb, gidx_b, dst8_b, isem_b)
        return carry

    lax.fori_loop(0, NCHUNK // 2, body, 0)
    pltpu.make_async_copy(gidx_hbm.at[pl.ds(0, CB)], gidx_b, isem_b).wait()
    pltpu.make_async_copy(dst8_hbm.at[pl.ds(0, CB)], dst8_b, isem_b).wait()
    for j in range(NG):
        pltpu.make_async_copy(
            hp8_hbm.at[gidx_a.at[pl.ds(j * GSUB, GSUB)]],
            rows_a.at[pl.ds(j * GSUB, GSUB)], gsem_a).wait()
    ob = w * ACC_TOT
    pltpu.sync_copy(acc0, out_hbm.at[pl.ds(ob, B1)])
    pltpu.sync_copy(acc1, out_hbm.at[pl.ds(ob + B1, B2 - B1)])
    pltpu.sync_copy(acc2, out_hbm.at[pl.ds(ob + B2, ACC_TOT - B2)])


_segmax = pl.kernel(
    _segmax_body,
    out_type=jax.ShapeDtypeStruct((NW * N * CPW,), jnp.float32),
    mesh=plsc.VectorSubcoreMesh(core_axis_name="c", subcore_axis_name="s",
                                num_cores=NC, num_subcores=NS),
    scratch_types=[
        pltpu.VMEM((CB,), jnp.int32),
        pltpu.VMEM((CB,), jnp.int32),
        pltpu.VMEM((CB, CPW), jnp.float32),
        pltpu.VMEM((CB,), jnp.int32),
        pltpu.VMEM((CB,), jnp.int32),
        pltpu.VMEM((CB, CPW), jnp.float32),
        pltpu.VMEM((Q0 * CPW,), jnp.float32),
        pltpu.VMEM((Q1 * CPW,), jnp.float32),
        pltpu.VMEM((Q2 * CPW,), jnp.float32),
        pltpu.VMEM((CB + 8,), jnp.int32),
        pltpu.VMEM((CB + 8,), jnp.int32),
        pltpu.VMEM((CB + 8,), jnp.int32),
        pltpu.SemaphoreType.DMA,
        pltpu.SemaphoreType.DMA,
        pltpu.SemaphoreType.DMA,
        pltpu.SemaphoreType.DMA,
    ],
    compiler_params=pltpu.CompilerParams(needs_layout_passes=False,
                                         use_tc_tiling_on_sc=False),
)


def _sage_layer(h, gidx32, dst8, WpT, bp, WsT, WnT, bs, act):
    hp, s = _mm_dual(h, WpT, bp, WsT, bs)
    hp8 = hp.reshape(N * NW, CPW)
    agg32 = _segmax(hp8, gidx32, dst8)
    agg = agg32.reshape(NW, N, CPW).transpose(1, 0, 2).reshape(N, D)
    return _mm_out(s, agg, WnT, act)


def kernel(x, edge_index, W_pool1, b_pool1, W_self1, W_neigh1, bias1,
           W_pool2, b_pool2, W_self2, W_neigh2, bias2):
    src = edge_index[0]
    dst = edge_index[1]
    gidx32 = src * NW
    dst8 = dst * CPW
    h = _sage_layer(x, gidx32, dst8, W_pool1.T, b_pool1, W_self1.T,
                    W_neigh1.T, bias1, True)
    h = _sage_layer(h, gidx32, dst8, W_pool2.T, b_pool2, W_self2.T,
                    W_neigh2.T, bias2, False)
    return h


# speedup vs baseline: 1.7990x; 1.0191x over previous
import functools

import jax
import jax.numpy as jnp
from jax import lax
from jax.experimental import pallas as pl
from jax.experimental.pallas import tpu as pltpu
from jax.experimental.pallas import tpu_sc as plsc

N = 10000
E = 160000
D = 256

NC = 2
NS = 16
NW = NC * NS
CPW = D // NW

CB = 640
NCHUNK = E // CB
GSUB = 128
NG = CB // GSUB

ROWBLK = 1000


def _mm_dual_body(x_ref, wp_ref, bp_ref, ws_ref, bs_ref, hp_ref, s_ref):
    xb = x_ref[...]
    hp = jnp.dot(xb, wp_ref[...], preferred_element_type=jnp.float32)
    hp_ref[...] = jnp.maximum(hp + bp_ref[...], 0.0)
    s = jnp.dot(xb, ws_ref[...], preferred_element_type=jnp.float32)
    s_ref[...] = s + bs_ref[...]


def _mm_dual(h, WpT, bp, WsT, bs):
    return pl.pallas_call(
        _mm_dual_body,
        grid=(N // ROWBLK,),
        in_specs=[
            pl.BlockSpec((ROWBLK, D), lambda i: (i, 0)),
            pl.BlockSpec((D, D), lambda i: (0, 0)),
            pl.BlockSpec((1, D), lambda i: (0, 0)),
            pl.BlockSpec((D, D), lambda i: (0, 0)),
            pl.BlockSpec((1, D), lambda i: (0, 0)),
        ],
        out_specs=[
            pl.BlockSpec((ROWBLK, D), lambda i: (i, 0)),
            pl.BlockSpec((ROWBLK, D), lambda i: (i, 0)),
        ],
        out_shape=[jax.ShapeDtypeStruct((N, D), jnp.float32)] * 2,
    )(h, WpT, bp.reshape(1, D), WsT, bs.reshape(1, D))


def _mm_out_body(act, s_ref, agg_ref, wn_ref, o_ref):
    o = s_ref[...] + jnp.dot(agg_ref[...], wn_ref[...],
                             preferred_element_type=jnp.float32)
    if act:
        o = jnp.tanh(o)
    o_ref[...] = o


def _mm_out(s, agg, WnT, act):
    return pl.pallas_call(
        functools.partial(_mm_out_body, act),
        grid=(N // ROWBLK,),
        in_specs=[
            pl.BlockSpec((ROWBLK, D), lambda i: (i, 0)),
            pl.BlockSpec((ROWBLK, D), lambda i: (i, 0)),
            pl.BlockSpec((D, D), lambda i: (0, 0)),
        ],
        out_specs=pl.BlockSpec((ROWBLK, D), lambda i: (i, 0)),
        out_shape=jax.ShapeDtypeStruct((N, D), jnp.float32),
    )(s, agg, WnT)


Q0 = 3334
Q1 = 3334
Q2 = N - Q0 - Q1
B1 = Q0 * CPW
B2 = (Q0 + Q1) * CPW
ACC_TOT = N * CPW


def _segmax_body(hp8_hbm, gidx_hbm, dst8_hbm, out_hbm,
                 gidx_a, dst8_a, rows_a, gidx_b, dst8_b, rows_b,
                 acc0, acc1, acc2, stream0, stream1, stream2,
                 isem_a, isem_b, gsem_a, gsem_b):
    w = lax.axis_index("s") * NC + lax.axis_index("c")

    iota = lax.iota(jnp.int32, 16)
    colpat = jnp.bitwise_and(iota, 7)
    pairsel = jnp.right_shift(iota, 3)
    perm8 = jnp.bitwise_xor(iota, 8)
    wvec = jnp.full((16,), 0, jnp.int32) + w
    zeros16 = jnp.zeros((16,), jnp.float32)

    for acc, q in ((acc0, Q0), (acc1, Q1), (acc2, Q2)):
        def zbody(i, carry, acc=acc):
            acc[pl.ds(i * 16, 16)] = zeros16
            return carry

        lax.fori_loop(0, (q * CPW) // 16, zbody, 0)

    def fire_idx(c, gidx_v, dst8_v, isem):
        e0 = c * CB
        cp1 = pltpu.async_copy(gidx_hbm.at[pl.ds(e0, CB)], gidx_v, isem)
        cp2 = pltpu.async_copy(dst8_hbm.at[pl.ds(e0, CB)], dst8_v, isem)
        return cp1, cp2

    def addw_fire_rows(gidx_v, rows_v, gsem):
        for i in range(CB // 16):
            sl = pl.ds(i * 16, 16)
            gidx_v[sl] = gidx_v[sl] + wvec
        return [
            pltpu.async_copy(hp8_hbm.at[gidx_v.at[pl.ds(j * GSUB, GSUB)]],
                             rows_v.at[pl.ds(j * GSUB, GSUB)], gsem)
            for j in range(NG)
        ]

    bcast15 = jnp.full((16,), 15, jnp.int32)

    def bclast(x):
        return x.at[bcast15].get(mode="promise_in_bounds")

    def pair_loop(dst8_v, rows_v, after_compact=None):
        def grp_body(i, offs):
            off0, off1, off2 = offs
            d8 = dst8_v[pl.ds(i * 16, 16)]
            ep17 = lax.shift_left(iota + i * 16, 17)
            c0 = d8 < B1
            c1 = d8 < B2
            m1 = c1 & (~c0)
            m2 = ~c1
            new = []
            for stream, acc_base, mq, off in ((stream0, 0, c0, off0),
                                              (stream1, B1, m1, off1),
                                              (stream2, B2, m2, off2)):
                entry = (d8 - acc_base) | ep17
                cum = plsc.cumsum(mq.astype(jnp.int32))
                tidx = off + cum - 1
                plsc.store_scatter(stream, [tidx], entry, mask=mq)
                new.append(off + bclast(cum))
            return tuple(new)

        zoff = jnp.zeros((16,), jnp.int32)
        off0, off1, off2 = lax.fori_loop(0, CB // 16, grp_body,
                                         (zoff, zoff, zoff), unroll=4)
        if after_compact is not None:
            after_compact()

        streams = (stream0, stream1, stream2)
        accs = (acc0, acc1, acc2)
        lens = (off0, off1, off2)
        maxlen = jnp.maximum(jnp.maximum(off0, off1), off2)
        nit = (jnp.max(maxlen) + 1) >> 1

        def fetch(si, t):
            epat = pairsel + t * 2
            mt = epat < lens[si]
            entry = plsc.load_gather(streams[si], [epat])
            entry = jnp.where(mt, entry, 0x1FFFF)
            d8c = entry & 0x1FFFF
            epos = lax.shift_right_logical(entry, 17)
            r = plsc.load_gather(rows_v, [epos, colpat])
            dswp = d8c.at[perm8].get(mode="promise_in_bounds",
                                     unique_indices=True)
            rswp = r.at[perm8].get(mode="promise_in_bounds",
                                   unique_indices=True)
            rpre = jnp.where(d8c == dswp, jnp.maximum(r, rswp), r)
            return d8c + colpat, rpre

        def rmw(si, t, fidx, rpre):
            mt = (pairsel + t * 2) < lens[si]
            a = plsc.load_gather(accs[si], [fidx], mask=mt)
            plsc.store_scatter(accs[si], [fidx], jnp.maximum(a, rpre),
                               mask=mt)

        pre = [fetch(si, 0) for si in range(3)]
        carry0 = (pre[0][0], pre[0][1], pre[1][0], pre[1][1],
                  pre[2][0], pre[2][1])
        nit4 = (nit + 3) >> 2

        def pp_body(t4, carry2):
            t = t4 * 4
            cur = ((carry2[0], carry2[1]), (carry2[2], carry2[3]),
                   (carry2[4], carry2[5]))
            for s in range(4):
                nxt = [fetch(si, t + s + 1) for si in range(3)]
                for si in range(3):
                    rmw(si, t + s, *cur[si])
                cur = nxt
            return (cur[0][0], cur[0][1], cur[1][0], cur[1][1],
                    cur[2][0], cur[2][1])

        lax.fori_loop(0, nit4, pp_body, carry0)

    last = NCHUNK - 1

    ia = fire_idx(0, gidx_a, dst8_a, isem_a)
    ib = fire_idx(1, gidx_b, dst8_b, isem_b)
    ia[0].wait()
    ia[1].wait()
    ga = addw_fire_rows(gidx_a, rows_a, gsem_a)

    def body(cc, carry):
        ca = 2 * cc
        nca = jnp.minimum(ca + 2, last)
        pltpu.make_async_copy(gidx_hbm.at[pl.ds(0, CB)], gidx_b, isem_b).wait()
        pltpu.make_async_copy(dst8_hbm.at[pl.ds(0, CB)], dst8_b, isem_b).wait()
        addw_fire_rows(gidx_b, rows_b, gsem_b)
        for j in range(NG):
            pltpu.make_async_copy(
                hp8_hbm.at[gidx_a.at[pl.ds(j * GSUB, GSUB)]],
                rows_a.at[pl.ds(j * GSUB, GSUB)], gsem_a).wait()
        e0a = nca * CB
        pltpu.async_copy(gidx_hbm.at[pl.ds(e0a, CB)], gidx_a, isem_a)

        def refill_dst8_a():
            pltpu.async_copy(dst8_hbm.at[pl.ds(e0a, CB)], dst8_a, isem_a)

        pair_loop(dst8_a, rows_a, after_compact=refill_dst8_a)
        for j in range(NG):
            pltpu.make_async_copy(
                hp8_hbm.at[gidx_b.at[pl.ds(j * GSUB, GSUB)]],
                rows_b.at[pl.ds(j * GSUB, GSUB)], gsem_b).wait()
        pltpu.make_async_copy(gidx_hbm.at[pl.ds(0, CB)], gidx_a, isem_a).wait()
        pltpu.make_async_copy(dst8_hbm.at[pl.ds(0, CB)], dst8_a, isem_a).wait()
        addw_fire_rows(gidx_a, rows_a, gsem_a)
        pair_loop(dst8_b, rows_b)
        ncb = jnp.minimum(ca + 3, last)
        fire_idx(ncb, gidx_b, dst8_b, isem_b)
        return carry

    lax.fori_loop(0, NCHUNK // 2, body, 0)
    pltpu.make_async_copy(gidx_hbm.at[pl.ds(0, CB)], gidx_b, isem_b).wait()
    pltpu.make_async_copy(dst8_hbm.at[pl.ds(0, CB)], dst8_b, isem_b).wait()
    for j in range(NG):
        pltpu.make_async_copy(
            hp8_hbm.at[gidx_a.at[pl.ds(j * GSUB, GSUB)]],
            rows_a.at[pl.ds(j * GSUB, GSUB)], gsem_a).wait()
    ob = w * ACC_TOT
    pltpu.sync_copy(acc0, out_hbm.at[pl.ds(ob, B1)])
    pltpu.sync_copy(acc1, out_hbm.at[pl.ds(ob + B1, B2 - B1)])
    pltpu.sync_copy(acc2, out_hbm.at[pl.ds(ob + B2, ACC_TOT - B2)])


_segmax = pl.kernel(
    _segmax_body,
    out_type=jax.ShapeDtypeStruct((NW * N * CPW,), jnp.float32),
    mesh=plsc.VectorSubcoreMesh(core_axis_name="c", subcore_axis_name="s",
                                num_cores=NC, num_subcores=NS),
    scratch_types=[
        pltpu.VMEM((CB,), jnp.int32),
        pltpu.VMEM((CB,), jnp.int32),
        pltpu.VMEM((CB, CPW), jnp.float32),
        pltpu.VMEM((CB,), jnp.int32),
        pltpu.VMEM((CB,), jnp.int32),
        pltpu.VMEM((CB, CPW), jnp.float32),
        pltpu.VMEM((Q0 * CPW,), jnp.float32),
        pltpu.VMEM((Q1 * CPW,), jnp.float32),
        pltpu.VMEM((Q2 * CPW,), jnp.float32),
        pltpu.VMEM((CB + 16,), jnp.int32),
        pltpu.VMEM((CB + 16,), jnp.int32),
        pltpu.VMEM((CB + 16,), jnp.int32),
        pltpu.SemaphoreType.DMA,
        pltpu.SemaphoreType.DMA,
        pltpu.SemaphoreType.DMA,
        pltpu.SemaphoreType.DMA,
    ],
    compiler_params=pltpu.CompilerParams(needs_layout_passes=False,
                                         use_tc_tiling_on_sc=False),
)


def _sage_layer(h, gidx32, dst8, WpT, bp, WsT, WnT, bs, act):
    hp, s = _mm_dual(h, WpT, bp, WsT, bs)
    hp8 = hp.reshape(N * NW, CPW)
    agg32 = _segmax(hp8, gidx32, dst8)
    agg = agg32.reshape(NW, N, CPW).transpose(1, 0, 2).reshape(N, D)
    return _mm_out(s, agg, WnT, act)


def kernel(x, edge_index, W_pool1, b_pool1, W_self1, W_neigh1, bias1,
           W_pool2, b_pool2, W_self2, W_neigh2, bias2):
    src = edge_index[0]
    dst = edge_index[1]
    gidx32 = src * NW
    dst8 = dst * CPW
    h = _sage_layer(x, gidx32, dst8, W_pool1.T, b_pool1, W_self1.T,
                    W_neigh1.T, bias1, True)
    h = _sage_layer(h, gidx32, dst8, W_pool2.T, b_pool2, W_self2.T,
                    W_neigh2.T, bias2, False)
    return h
